# baseline (speedup 1.0000x reference)
import functools

import jax
import jax.numpy as jnp
from jax import lax
from jax.experimental import pallas as pl
from jax.experimental.pallas import tpu as pltpu
from jax.experimental.pallas import tpu_sc as plsc

NC = 2
NS = 16
NW = NC * NS
CHUNK = 64


def _gather_body(chunk_base, bpw, depth,
                 uidx, midx, utab, mtab, uout, mout, idx_v, rows_v, gsem, wsem):
    nch = bpw // CHUNK
    wid = lax.axis_index("s") * NC + lax.axis_index("c")
    base = wid * bpw

    pltpu.sync_copy(uidx.at[pl.ds(chunk_base + base, bpw)], idx_v.at[0])
    pltpu.sync_copy(midx.at[pl.ds(chunk_base + base, bpw)], idx_v.at[1])

    tasks = [(t, j, tab, out)
             for t, (tab, out) in enumerate(((utab, uout), (mtab, mout)))
             for j in range(nch)]
    n = len(tasks)

    def fire_gather(k):
        t, j, tab, _ = tasks[k]
        return pltpu.async_copy(
            tab.at[idx_v.at[t, pl.ds(j * CHUNK, CHUNK)]],
            rows_v.at[k % depth], gsem.at[k % depth])

    gathers = [None] * n
    writes = [None] * n
    for k in range(min(depth, n)):
        gathers[k] = fire_gather(k)
    for k in range(n):
        t, j, _, out = tasks[k]
        gathers[k].wait()
        writes[k] = pltpu.async_copy(
            rows_v.at[k % depth],
            out.at[pl.ds(base + j * CHUNK, CHUNK)], wsem.at[k % depth])
        kn = k + depth
        if kn < n:
            writes[k].wait()
            gathers[kn] = fire_gather(kn)
    for k in range(max(0, n - depth), n):
        writes[k].wait()


def _mlp_body(aliased, xu_hbm, xm_hbm, w1_ref, b1_ref, w2_ref, b2_ref, w3_ref,
              b3_ref, *rest):
    if aliased:
        rest = rest[1:]
    out_ref, xu_buf, xm_buf, usem, msem = rest
    i = pl.program_id(0)
    nsteps = pl.num_programs(0)
    nbuf, BLK, D = xu_buf.shape[0], xu_buf.shape[1], xu_buf.shape[2]
    pf = nbuf - 1

    def copies(step, slot):
        return (
            pltpu.make_async_copy(xu_hbm.at[pl.ds(step * BLK, BLK)],
                                  xu_buf.at[slot], usem.at[slot]),
            pltpu.make_async_copy(xm_hbm.at[pl.ds(step * BLK, BLK)],
                                  xm_buf.at[slot], msem.at[slot]),
        )

    @pl.when(i == 0)
    def _():
        for s in range(pf):
            if s < nsteps:
                for c in copies(s, s):
                    c.start()

    nxt = i + pf
    @pl.when(nxt < nsteps)
    def _():
        for c in copies(nxt, lax.rem(nxt, nbuf)):
            c.start()

    slot = lax.rem(i, nbuf)
    for c in copies(i, slot):
        c.wait()

    h = (jnp.dot(xu_buf[slot], w1_ref[:D], preferred_element_type=jnp.float32)
         + jnp.dot(xm_buf[slot], w1_ref[D:], preferred_element_type=jnp.float32)
         + b1_ref[...])
    h = jnp.maximum(h, 0.0)
    h2t = lax.dot_general(w2_ref[...], h, (((0,), (1,)), ((), ())),
                          preferred_element_type=jnp.float32)
    h2t = jnp.maximum(h2t + b2_ref[...], 0.0)
    ot = lax.dot_general(w3_ref[...], h2t, (((0,), (0,)), ((), ())),
                         preferred_element_type=jnp.float32)
    ot = jnp.maximum(ot + b3_ref[...], 0.0)
    out_ref[...] = ot[0]


def kernel(users, movies, user_table, movie_table, W1, b1, W2, b2, W3, b3):
    B = users.shape[0]
    D = user_table.shape[1]
    NCHK = 2
    Bc = B // NCHK
    bpw = Bc // NW
    depth = min(8, 2 * (bpw // CHUNK))

    uidx = users.astype(jnp.int32)
    midx = movies.astype(jnp.int32)

    mesh = plsc.VectorSubcoreMesh(core_axis_name="c", subcore_axis_name="s")

    def make_gather(chunk_base):
        return pl.kernel(
            functools.partial(_gather_body, chunk_base, bpw, depth),
            out_type=[
                jax.ShapeDtypeStruct((Bc, D), jnp.float32),
                jax.ShapeDtypeStruct((Bc, D), jnp.float32),
            ],
            mesh=mesh,
            scratch_types=[
                pltpu.VMEM((2, bpw), jnp.int32),
                pltpu.VMEM((depth, CHUNK, D), jnp.float32),
                pltpu.SemaphoreType.DMA((depth,)),
                pltpu.SemaphoreType.DMA((depth,)),
            ],
        )

    BLK = 2048
    NBUF = 4
    nsteps = Bc // BLK

    def make_mlp(c, aliased):
        in_specs = [
            pl.BlockSpec(memory_space=pl.ANY),
            pl.BlockSpec(memory_space=pl.ANY),
            pl.BlockSpec((2 * D, 64), lambda i: (0, 0)),
            pl.BlockSpec((64,), lambda i: (0,)),
            pl.BlockSpec((64, 16), lambda i: (0, 0)),
            pl.BlockSpec((16, 1), lambda i: (0, 0)),
            pl.BlockSpec((16, 1), lambda i: (0, 0)),
            pl.BlockSpec((1, 1), lambda i: (0, 0)),
        ]
        if aliased:
            in_specs.append(pl.BlockSpec(memory_space=pl.ANY))
        return pl.pallas_call(
            functools.partial(_mlp_body, aliased),
            grid=(nsteps,),
            in_specs=in_specs,
            out_specs=pl.BlockSpec((BLK,), lambda i, c=c: (i + c * nsteps,)),
            out_shape=jax.ShapeDtypeStruct((B,), jnp.float32),
            input_output_aliases={8: 0} if aliased else {},
            scratch_shapes=[
                pltpu.VMEM((NBUF, BLK, D), jnp.float32),
                pltpu.VMEM((NBUF, BLK, D), jnp.float32),
                pltpu.SemaphoreType.DMA((NBUF,)),
                pltpu.SemaphoreType.DMA((NBUF,)),
            ],
            compiler_params=pltpu.CompilerParams(
                dimension_semantics=("arbitrary",)),
        )

    acc = None
    for c in range(NCHK):
        ue, me = make_gather(c * Bc)(uidx, midx, user_table, movie_table)
        ue = pltpu.with_memory_space_constraint(ue, pltpu.MemorySpace.HBM)
        me = pltpu.with_memory_space_constraint(me, pltpu.MemorySpace.HBM)
        args = (ue, me, W1, b1, W2, b2.reshape(16, 1), W3, b3.reshape(1, 1))
        acc = make_mlp(c, c > 0)(*args, *(() if c == 0 else (acc,)))
    return acc


# speedup vs baseline: 1.0129x; 1.0129x over previous
import functools

import jax
import jax.numpy as jnp
from jax import lax
from jax.experimental import pallas as pl
from jax.experimental.pallas import tpu as pltpu
from jax.experimental.pallas import tpu_sc as plsc

NC = 2
NS = 16
NW = NC * NS
CHUNK = 128


def _gather_body(chunk_base, bpw, depth,
                 uidx, midx, utab, mtab, uout, mout, idx_v, rows_v, gsem, wsem):
    nch = bpw // CHUNK
    wid = lax.axis_index("s") * NC + lax.axis_index("c")
    base = wid * bpw

    pltpu.sync_copy(uidx.at[pl.ds(chunk_base + base, bpw)], idx_v.at[0])
    pltpu.sync_copy(midx.at[pl.ds(chunk_base + base, bpw)], idx_v.at[1])

    tasks = [(t, j, tab, out)
             for t, (tab, out) in enumerate(((utab, uout), (mtab, mout)))
             for j in range(nch)]
    n = len(tasks)

    def fire_gather(k):
        t, j, tab, _ = tasks[k]
        return pltpu.async_copy(
            tab.at[idx_v.at[t, pl.ds(j * CHUNK, CHUNK)]],
            rows_v.at[k % depth], gsem.at[k % depth])

    gathers = [None] * n
    writes = [None] * n
    for k in range(min(depth, n)):
        gathers[k] = fire_gather(k)
    for k in range(n):
        t, j, _, out = tasks[k]
        gathers[k].wait()
        writes[k] = pltpu.async_copy(
            rows_v.at[k % depth],
            out.at[pl.ds(base + j * CHUNK, CHUNK)], wsem.at[k % depth])
        kn = k + depth
        if kn < n:
            writes[k].wait()
            gathers[kn] = fire_gather(kn)
    for k in range(max(0, n - depth), n):
        writes[k].wait()


def _mlp_body(aliased, xu_hbm, xm_hbm, w1_ref, b1_ref, w2_ref, b2_ref, w3_ref,
              b3_ref, *rest):
    if aliased:
        rest = rest[1:]
    out_ref, xu_buf, xm_buf, usem, msem = rest
    i = pl.program_id(0)
    nsteps = pl.num_programs(0)
    nbuf, BLK, D = xu_buf.shape[0], xu_buf.shape[1], xu_buf.shape[2]
    pf = nbuf - 1

    def copies(step, slot):
        return (
            pltpu.make_async_copy(xu_hbm.at[pl.ds(step * BLK, BLK)],
                                  xu_buf.at[slot], usem.at[slot]),
            pltpu.make_async_copy(xm_hbm.at[pl.ds(step * BLK, BLK)],
                                  xm_buf.at[slot], msem.at[slot]),
        )

    @pl.when(i == 0)
    def _():
        for s in range(pf):
            if s < nsteps:
                for c in copies(s, s):
                    c.start()

    nxt = i + pf
    @pl.when(nxt < nsteps)
    def _():
        for c in copies(nxt, lax.rem(nxt, nbuf)):
            c.start()

    slot = lax.rem(i, nbuf)
    for c in copies(i, slot):
        c.wait()

    h = (jnp.dot(xu_buf[slot], w1_ref[:D], preferred_element_type=jnp.float32)
         + jnp.dot(xm_buf[slot], w1_ref[D:], preferred_element_type=jnp.float32)
         + b1_ref[...])
    h = jnp.maximum(h, 0.0)
    h2t = lax.dot_general(w2_ref[...], h, (((0,), (1,)), ((), ())),
                          preferred_element_type=jnp.float32)
    h2t = jnp.maximum(h2t + b2_ref[...], 0.0)
    ot = lax.dot_general(w3_ref[...], h2t, (((0,), (0,)), ((), ())),
                         preferred_element_type=jnp.float32)
    ot = jnp.maximum(ot + b3_ref[...], 0.0)
    out_ref[...] = ot[0]


def kernel(users, movies, user_table, movie_table, W1, b1, W2, b2, W3, b3):
    B = users.shape[0]
    D = user_table.shape[1]
    NCHK = 2
    Bc = B // NCHK
    bpw = Bc // NW
    depth = min(7, 2 * (bpw // CHUNK))

    uidx = users.astype(jnp.int32)
    midx = movies.astype(jnp.int32)

    mesh = plsc.VectorSubcoreMesh(core_axis_name="c", subcore_axis_name="s")

    def make_gather(chunk_base):
        return pl.kernel(
            functools.partial(_gather_body, chunk_base, bpw, depth),
            out_type=[
                jax.ShapeDtypeStruct((Bc, D), jnp.float32),
                jax.ShapeDtypeStruct((Bc, D), jnp.float32),
            ],
            mesh=mesh,
            scratch_types=[
                pltpu.VMEM((2, bpw), jnp.int32),
                pltpu.VMEM((depth, CHUNK, D), jnp.float32),
                pltpu.SemaphoreType.DMA((depth,)),
                pltpu.SemaphoreType.DMA((depth,)),
            ],
        )

    BLK = 2048
    NBUF = 4
    nsteps = Bc // BLK

    def make_mlp(c, aliased):
        in_specs = [
            pl.BlockSpec(memory_space=pl.ANY),
            pl.BlockSpec(memory_space=pl.ANY),
            pl.BlockSpec((2 * D, 64), lambda i: (0, 0)),
            pl.BlockSpec((64,), lambda i: (0,)),
            pl.BlockSpec((64, 16), lambda i: (0, 0)),
            pl.BlockSpec((16, 1), lambda i: (0, 0)),
            pl.BlockSpec((16, 1), lambda i: (0, 0)),
            pl.BlockSpec((1, 1), lambda i: (0, 0)),
        ]
        if aliased:
            in_specs.append(pl.BlockSpec(memory_space=pl.ANY))
        return pl.pallas_call(
            functools.partial(_mlp_body, aliased),
            grid=(nsteps,),
            in_specs=in_specs,
            out_specs=pl.BlockSpec((BLK,), lambda i, c=c: (i + c * nsteps,)),
            out_shape=jax.ShapeDtypeStruct((B,), jnp.float32),
            input_output_aliases={8: 0} if aliased else {},
            scratch_shapes=[
                pltpu.VMEM((NBUF, BLK, D), jnp.float32),
                pltpu.VMEM((NBUF, BLK, D), jnp.float32),
                pltpu.SemaphoreType.DMA((NBUF,)),
                pltpu.SemaphoreType.DMA((NBUF,)),
            ],
            compiler_params=pltpu.CompilerParams(
                dimension_semantics=("arbitrary",)),
        )

    gather = make_gather(0)
    acc = None
    for c in range(NCHK):
        ue, me = gather(uidx[c * Bc:(c + 1) * Bc], midx[c * Bc:(c + 1) * Bc],
                        user_table, movie_table)
        ue = pltpu.with_memory_space_constraint(ue, pltpu.MemorySpace.HBM)
        me = pltpu.with_memory_space_constraint(me, pltpu.MemorySpace.HBM)
        args = (ue, me, W1, b1, W2, b2.reshape(16, 1), W3, b3.reshape(1, 1))
        acc = make_mlp(c, c > 0)(*args, *(() if c == 0 else (acc,)))
    return acc


# speedup vs baseline: 1.0223x; 1.0093x over previous
import functools

import jax
import jax.numpy as jnp
from jax import lax
from jax.experimental import pallas as pl
from jax.experimental.pallas import tpu as pltpu
from jax.experimental.pallas import tpu_sc as plsc

NC = 2
NS = 16
NW = NC * NS
CHUNK = 128


def _gather_body(chunk_base, bpw, depth,
                 uidx, midx, utab, mtab, uout, mout, idx_v, rows_v, gsem, wsem):
    nch = bpw // CHUNK
    wid = lax.axis_index("s") * NC + lax.axis_index("c")
    base = wid * bpw

    pltpu.sync_copy(uidx.at[pl.ds(chunk_base + base, bpw)], idx_v.at[0])
    pltpu.sync_copy(midx.at[pl.ds(chunk_base + base, bpw)], idx_v.at[1])

    tasks = [(t, j, tab, out)
             for t, (tab, out) in enumerate(((utab, uout), (mtab, mout)))
             for j in range(nch)]
    n = len(tasks)

    def fire_gather(k):
        t, j, tab, _ = tasks[k]
        return pltpu.async_copy(
            tab.at[idx_v.at[t, pl.ds(j * CHUNK, CHUNK)]],
            rows_v.at[k % depth], gsem.at[k % depth])

    gathers = [None] * n
    writes = [None] * n
    for k in range(min(depth, n)):
        gathers[k] = fire_gather(k)
    for k in range(n):
        t, j, _, out = tasks[k]
        gathers[k].wait()
        writes[k] = pltpu.async_copy(
            rows_v.at[k % depth],
            out.at[pl.ds(base + j * CHUNK, CHUNK)], wsem.at[k % depth])
        kn = k + depth
        if kn < n:
            writes[k].wait()
            gathers[kn] = fire_gather(kn)
    for k in range(max(0, n - depth), n):
        writes[k].wait()


def _mlp_body(aliased, xu_hbm, xm_hbm, w1_ref, b1_ref, w2_ref, b2_ref, w3_ref,
              b3_ref, *rest):
    if aliased:
        rest = rest[1:]
    out_ref, xu_buf, xm_buf, usem, msem = rest
    i = pl.program_id(0)
    nsteps = pl.num_programs(0)
    nbuf, BLK, D = xu_buf.shape[0], xu_buf.shape[1], xu_buf.shape[2]
    pf = nbuf - 1

    H = BLK // 2

    def copies(step, slot):
        return (
            pltpu.make_async_copy(xu_hbm.at[pl.ds(step * BLK, H)],
                                  xu_buf.at[slot, pl.ds(0, H)],
                                  usem.at[slot, 0]),
            pltpu.make_async_copy(xu_hbm.at[pl.ds(step * BLK + H, H)],
                                  xu_buf.at[slot, pl.ds(H, H)],
                                  usem.at[slot, 1]),
            pltpu.make_async_copy(xm_hbm.at[pl.ds(step * BLK, H)],
                                  xm_buf.at[slot, pl.ds(0, H)],
                                  msem.at[slot, 0]),
            pltpu.make_async_copy(xm_hbm.at[pl.ds(step * BLK + H, H)],
                                  xm_buf.at[slot, pl.ds(H, H)],
                                  msem.at[slot, 1]),
        )

    @pl.when(i == 0)
    def _():
        for s in range(pf):
            if s < nsteps:
                for c in copies(s, s):
                    c.start()

    nxt = i + pf
    @pl.when(nxt < nsteps)
    def _():
        for c in copies(nxt, lax.rem(nxt, nbuf)):
            c.start()

    slot = lax.rem(i, nbuf)
    for c in copies(i, slot):
        c.wait()

    h = (jnp.dot(xu_buf[slot], w1_ref[:D], preferred_element_type=jnp.float32)
         + jnp.dot(xm_buf[slot], w1_ref[D:], preferred_element_type=jnp.float32)
         + b1_ref[...])
    h = jnp.maximum(h, 0.0)
    h2t = lax.dot_general(w2_ref[...], h, (((0,), (1,)), ((), ())),
                          preferred_element_type=jnp.float32)
    h2t = jnp.maximum(h2t + b2_ref[...], 0.0)
    ot = lax.dot_general(w3_ref[...], h2t, (((0,), (0,)), ((), ())),
                         preferred_element_type=jnp.float32)
    ot = jnp.maximum(ot + b3_ref[...], 0.0)
    out_ref[...] = ot[0]


def kernel(users, movies, user_table, movie_table, W1, b1, W2, b2, W3, b3):
    B = users.shape[0]
    D = user_table.shape[1]
    NCHK = 2
    Bc = B // NCHK
    bpw = Bc // NW
    depth = min(7, 2 * (bpw // CHUNK))

    uidx = users.astype(jnp.int32)
    midx = movies.astype(jnp.int32)

    mesh = plsc.VectorSubcoreMesh(core_axis_name="c", subcore_axis_name="s")

    def make_gather(chunk_base):
        return pl.kernel(
            functools.partial(_gather_body, chunk_base, bpw, depth),
            out_type=[
                jax.ShapeDtypeStruct((Bc, D), jnp.float32),
                jax.ShapeDtypeStruct((Bc, D), jnp.float32),
            ],
            mesh=mesh,
            scratch_types=[
                pltpu.VMEM((2, bpw), jnp.int32),
                pltpu.VMEM((depth, CHUNK, D), jnp.float32),
                pltpu.SemaphoreType.DMA((depth,)),
                pltpu.SemaphoreType.DMA((depth,)),
            ],
        )

    BLK = 2048
    NBUF = 4
    nsteps = Bc // BLK

    def make_mlp(c, aliased):
        in_specs = [
            pl.BlockSpec(memory_space=pl.ANY),
            pl.BlockSpec(memory_space=pl.ANY),
            pl.BlockSpec((2 * D, 64), lambda i: (0, 0)),
            pl.BlockSpec((64,), lambda i: (0,)),
            pl.BlockSpec((64, 16), lambda i: (0, 0)),
            pl.BlockSpec((16, 1), lambda i: (0, 0)),
            pl.BlockSpec((16, 1), lambda i: (0, 0)),
            pl.BlockSpec((1, 1), lambda i: (0, 0)),
        ]
        if aliased:
            in_specs.append(pl.BlockSpec(memory_space=pl.ANY))
        return pl.pallas_call(
            functools.partial(_mlp_body, aliased),
            grid=(nsteps,),
            in_specs=in_specs,
            out_specs=pl.BlockSpec((BLK,), lambda i, c=c: (i + c * nsteps,)),
            out_shape=jax.ShapeDtypeStruct((B,), jnp.float32),
            input_output_aliases={8: 0} if aliased else {},
            scratch_shapes=[
                pltpu.VMEM((NBUF, BLK, D), jnp.float32),
                pltpu.VMEM((NBUF, BLK, D), jnp.float32),
                pltpu.SemaphoreType.DMA((NBUF, 2)),
                pltpu.SemaphoreType.DMA((NBUF, 2)),
            ],
            compiler_params=pltpu.CompilerParams(
                dimension_semantics=("arbitrary",)),
        )

    gather = make_gather(0)
    acc = None
    for c in range(NCHK):
        ue, me = gather(uidx[c * Bc:(c + 1) * Bc], midx[c * Bc:(c + 1) * Bc],
                        user_table, movie_table)
        ue = pltpu.with_memory_space_constraint(ue, pltpu.MemorySpace.HBM)
        me = pltpu.with_memory_space_constraint(me, pltpu.MemorySpace.HBM)
        args = (ue, me, W1, b1, W2, b2.reshape(16, 1), W3, b3.reshape(1, 1))
        acc = make_mlp(c, c > 0)(*args, *(() if c == 0 else (acc,)))
    return acc
